# baseline (speedup 1.0000x reference)
import jax
import jax.numpy as jnp
from jax import lax
from jax.experimental import pallas as pl
from jax.experimental.pallas import tpu as pltpu
from jax.experimental.pallas import tpu_sc as plsc

_NUM_CORES = 2
_NUM_SUBCORES = 16


def _sc_gather(table, idx):
    (B,) = idx.shape
    V, D = table.shape
    nw = _NUM_CORES * _NUM_SUBCORES
    b_per_w = B // nw
    g_per_w = b_per_w // 8

    table_t = table.T

    def body(table_hbm, idx_hbm, out_hbm, idx_v, off_v, rows_v, out_v, sem):
        wid = lax.axis_index("s") * _NUM_CORES + lax.axis_index("c")
        base = wid * b_per_w
        pltpu.sync_copy(idx_hbm.at[pl.ds(base, b_per_w)], idx_v)
        iota = lax.iota(jnp.int32, 16)
        for g in range(b_per_w // 16):
            sl = pl.ds(g * 16, 16)
            off_v[sl] = jax.lax.bitwise_and(idx_v[sl], jnp.int32(~127))
        for g in range(b_per_w // 16):
            ochunk = off_v[pl.ds(g * 16, 16)]
            for l in range(16):
                b = g * 16 + l
                off = pl.multiple_of(jnp.max(jnp.where(iota == l, ochunk, 0)), 128)
                pltpu.make_async_copy(
                    table_hbm.at[:, pl.ds(off, 128)], rows_v.at[b], sem
                ).start()
        for b in range(b_per_w):
            pltpu.make_async_copy(
                table_hbm.at[:, pl.ds(0, 128)], rows_v.at[0], sem
            ).wait()
        for g in range(b_per_w // 16):
            bvec = iota + g * 16
            sl = pl.ds(g * 16, 16)
            lvec = idx_v[sl] - off_v[sl]
            for d in range(D):
                dfull = jnp.full((16,), d, jnp.int32)
                vals = plsc.load_gather(rows_v, [bvec, dfull, lvec])
                plsc.store_scatter(
                    out_v,
                    [lax.shift_right_logical(bvec, 3), lax.rem(bvec, 8), dfull],
                    vals,
                )
        pltpu.sync_copy(out_v, out_hbm.at[pl.ds(wid * g_per_w, g_per_w)])

    mesh = plsc.VectorSubcoreMesh(core_axis_name="c", subcore_axis_name="s")
    out3 = pl.kernel(
        body,
        out_type=jax.ShapeDtypeStruct((B // 8, 8, D), jnp.float32),
        mesh=mesh,
        scratch_types=[
            pltpu.VMEM((b_per_w,), jnp.int32),
            pltpu.VMEM((b_per_w,), jnp.int32),
            pltpu.VMEM((b_per_w, D, 128), jnp.float32),
            pltpu.VMEM((g_per_w, 8, D), jnp.float32),
            pltpu.SemaphoreType.DMA,
        ],
        compiler_params=pltpu.CompilerParams(
            needs_layout_passes=False, disable_bounds_checks=True
        ),
    )(table_t, idx)
    return out3.reshape(B, D)


def _mm_body(wt_ref, embt_ref, out_ref):
    out_ref[...] = jnp.dot(
        wt_ref[...], embt_ref[...], preferred_element_type=jnp.float32
    )


def _tc_project_t(wt, embt, tile_v=4000):
    V, D = wt.shape
    B = embt.shape[1]
    return pl.pallas_call(
        _mm_body,
        grid=(V // tile_v,),
        in_specs=[
            pl.BlockSpec((tile_v, D), lambda j: (j, 0)),
            pl.BlockSpec((D, B), lambda j: (0, 0)),
        ],
        out_specs=pl.BlockSpec((tile_v, B), lambda j: (j, 0)),
        out_shape=jax.ShapeDtypeStruct((V, B), jnp.float32),
    )(wt, embt)


def kernel(input_data, embedding_table, fc_weight, fc_bias):
    emb = _sc_gather(embedding_table, input_data.astype(jnp.int32))
    B = emb.shape[0]
    w_aug = jnp.concatenate([fc_weight, fc_bias[None, :]], axis=0)
    embt_aug = jnp.concatenate(
        [emb.T, jnp.ones((1, B), jnp.float32)], axis=0
    )
    out_t = _tc_project_t(
        w_aug.T.astype(jnp.bfloat16),
        embt_aug.astype(jnp.bfloat16),
    )
    return out_t.T
